# Initial kernel scaffold; baseline (speedup 1.0000x reference)
import jax
import jax.numpy as jnp
from jax.experimental import pallas as pl


def kernel(visual_input, conv_weight, conv_bias):
    raise NotImplementedError("write your pallas kernel here")


# speedup vs baseline: 4.0153x; 4.0153x over previous
import functools

import jax
import jax.numpy as jnp
from jax.experimental import pallas as pl
from jax.experimental.pallas import tpu as pltpu


def _pool_matmul_kernel(x_ref, p_ref, w_ref, b_ref, o_ref, *, chunk):
    x = x_ref[...].astype(jnp.bfloat16)
    p = p_ref[...]
    n_chunks = x.shape[0] // (2 * chunk)
    pooled = jnp.concatenate(
        [
            jnp.dot(
                p,
                x[2 * chunk * c : 2 * chunk * (c + 1), :],
                preferred_element_type=jnp.float32,
            )
            for c in range(n_chunks)
        ],
        axis=0,
    ).astype(jnp.bfloat16)
    o_ref[...] = (
        jnp.dot(pooled, w_ref[...], preferred_element_type=jnp.float32)
        + b_ref[...]
    ).astype(o_ref.dtype)


def kernel(visual_input, conv_weight, conv_bias):
    B, T, C_in = visual_input.shape
    C_out = conv_weight.shape[0]
    ks = 2
    T_out = T // ks
    M = B * T_out
    dtype = visual_input.dtype

    x2d = visual_input.reshape(B * T, C_in)
    w = jnp.transpose(conv_weight[:, :, 0], (1, 0)).astype(jnp.bfloat16)
    bias = conv_bias.reshape(1, C_out).astype(jnp.float32)

    chunk = 256
    ii = jax.lax.broadcasted_iota(jnp.int32, (chunk, 2 * chunk), 0)
    jj = jax.lax.broadcasted_iota(jnp.int32, (chunk, 2 * chunk), 1)
    pmat = jnp.where((jj == 2 * ii) | (jj == 2 * ii + 1), 1.0 / ks, 0.0).astype(
        jnp.bfloat16
    )

    rt = 4096
    grid = (M // rt,)

    out2d = pl.pallas_call(
        functools.partial(_pool_matmul_kernel, chunk=chunk),
        out_shape=jax.ShapeDtypeStruct((M, C_out), dtype),
        grid_spec=pltpu.PrefetchScalarGridSpec(
            num_scalar_prefetch=0,
            grid=grid,
            in_specs=[
                pl.BlockSpec((ks * rt, C_in), lambda i: (i, 0)),
                pl.BlockSpec((chunk, 2 * chunk), lambda i: (0, 0)),
                pl.BlockSpec((C_in, C_out), lambda i: (0, 0)),
                pl.BlockSpec((1, C_out), lambda i: (0, 0)),
            ],
            out_specs=pl.BlockSpec((rt, C_out), lambda i: (i, 0)),
        ),
        compiler_params=pltpu.CompilerParams(
            dimension_semantics=("parallel",),
        ),
    )(x2d, pmat, w, bias)
    return out2d.reshape(B, T_out, C_out)


# speedup vs baseline: 4.0537x; 1.0096x over previous
import functools

import jax
import jax.numpy as jnp
import numpy as np
from jax.experimental import pallas as pl
from jax.experimental.pallas import tpu as pltpu


def _pool_matmul_kernel(x_ref, p_ref, w_ref, b_ref, o_ref, *, chunk):
    x = x_ref[...].astype(jnp.bfloat16)
    p = p_ref[...]
    n_chunks = x.shape[0] // (2 * chunk)
    pooled = jnp.concatenate(
        [
            jnp.dot(
                p,
                x[2 * chunk * c : 2 * chunk * (c + 1), :],
                preferred_element_type=jnp.float32,
            )
            for c in range(n_chunks)
        ],
        axis=0,
    ).astype(jnp.bfloat16)
    o_ref[...] = (
        jnp.dot(pooled, w_ref[...], preferred_element_type=jnp.float32)
        + b_ref[...]
    ).astype(o_ref.dtype)


def kernel(visual_input, conv_weight, conv_bias):
    B, T, C_in = visual_input.shape
    C_out = conv_weight.shape[0]
    ks = 2
    T_out = T // ks
    M = B * T_out
    dtype = visual_input.dtype

    x2d = visual_input.reshape(B * T, C_in)
    w = jnp.transpose(conv_weight[:, :, 0], (1, 0)).astype(jnp.bfloat16)
    bias = conv_bias.reshape(1, C_out).astype(jnp.float32)

    chunk = 256
    ii = np.arange(chunk)[:, None]
    jj = np.arange(2 * chunk)[None, :]
    pmat = jnp.asarray(
        ((jj == 2 * ii) | (jj == 2 * ii + 1)) * (1.0 / ks), jnp.bfloat16
    )

    rt = 4096
    grid = (M // rt,)

    out2d = pl.pallas_call(
        functools.partial(_pool_matmul_kernel, chunk=chunk),
        out_shape=jax.ShapeDtypeStruct((M, C_out), dtype),
        grid_spec=pltpu.PrefetchScalarGridSpec(
            num_scalar_prefetch=0,
            grid=grid,
            in_specs=[
                pl.BlockSpec((ks * rt, C_in), lambda i: (i, 0)),
                pl.BlockSpec((chunk, 2 * chunk), lambda i: (0, 0)),
                pl.BlockSpec((C_in, C_out), lambda i: (0, 0)),
                pl.BlockSpec((1, C_out), lambda i: (0, 0)),
            ],
            out_specs=pl.BlockSpec((rt, C_out), lambda i: (i, 0)),
        ),
        compiler_params=pltpu.CompilerParams(
            dimension_semantics=("parallel",),
        ),
    )(x2d, pmat, w, bias)
    return out2d.reshape(B, T_out, C_out)


# speedup vs baseline: 4.0618x; 1.0020x over previous
import functools

import jax
import jax.numpy as jnp
import numpy as np
from jax.experimental import pallas as pl
from jax.experimental.pallas import tpu as pltpu


def _pool_matmul_kernel(x_ref, p_ref, w_ref, b_ref, o_ref, *, chunk):
    x = x_ref[...].astype(jnp.bfloat16)
    p = p_ref[...]
    n_chunks = x.shape[0] // (2 * chunk)
    pooled = jnp.concatenate(
        [
            jnp.dot(
                p,
                x[2 * chunk * c : 2 * chunk * (c + 1), :],
                preferred_element_type=jnp.float32,
            )
            for c in range(n_chunks)
        ],
        axis=0,
    ).astype(jnp.bfloat16)
    w = w_ref[...].astype(jnp.bfloat16)
    o_ref[...] = (
        jax.lax.dot_general(
            pooled,
            w,
            dimension_numbers=(((1,), (1,)), ((), ())),
            preferred_element_type=jnp.float32,
        )
        + b_ref[...]
    ).astype(o_ref.dtype)


def kernel(visual_input, conv_weight, conv_bias):
    B, T, C_in = visual_input.shape
    C_out = conv_weight.shape[0]
    ks = 2
    T_out = T // ks
    M = B * T_out
    dtype = visual_input.dtype

    x2d = visual_input.reshape(B * T, C_in)
    w = conv_weight[:, :, 0]
    bias = conv_bias.reshape(1, C_out).astype(jnp.float32)

    chunk = 256
    ii = np.arange(chunk)[:, None]
    jj = np.arange(2 * chunk)[None, :]
    pmat = jnp.asarray(
        ((jj == 2 * ii) | (jj == 2 * ii + 1)) * (1.0 / ks), jnp.bfloat16
    )

    rt = 4096
    grid = (M // rt,)

    out2d = pl.pallas_call(
        functools.partial(_pool_matmul_kernel, chunk=chunk),
        out_shape=jax.ShapeDtypeStruct((M, C_out), dtype),
        grid_spec=pltpu.PrefetchScalarGridSpec(
            num_scalar_prefetch=0,
            grid=grid,
            in_specs=[
                pl.BlockSpec((ks * rt, C_in), lambda i: (i, 0)),
                pl.BlockSpec((chunk, 2 * chunk), lambda i: (0, 0)),
                pl.BlockSpec((C_out, C_in), lambda i: (0, 0)),
                pl.BlockSpec((1, C_out), lambda i: (0, 0)),
            ],
            out_specs=pl.BlockSpec((rt, C_out), lambda i: (i, 0)),
        ),
        compiler_params=pltpu.CompilerParams(
            dimension_semantics=("parallel",),
        ),
    )(x2d, pmat, w, bias)
    return out2d.reshape(B, T_out, C_out)
